# Initial kernel scaffold; baseline (speedup 1.0000x reference)
import jax
import jax.numpy as jnp
from jax.experimental import pallas as pl


def kernel(x, edge_index, W1_0, W1_1, b1, W2_0, W2_1, b2):
    raise NotImplementedError("write your pallas kernel here")


# speedup vs baseline: 13.6750x; 13.6750x over previous
import functools

import jax
import jax.numpy as jnp
from jax import lax
from jax.experimental import pallas as pl
from jax.experimental.pallas import tpu as pltpu
from jax.experimental.pallas import tpu_sc as plsc

N_NODES = 10000
N_EDGES = 320000
D_FEAT = 128
D_HID = 16

NC = 2
NS = 16
NW = NC * NS
L = 16

NP = 10240
RPT = NP // NS
EPW = N_EDGES // NW
CHUNK = 80
NITER = EPW // CHUNK

_mesh = plsc.VectorSubcoreMesh(core_axis_name="c", subcore_axis_name="s")


def _zero_fill(ztile, acc, sid):
    for r in range(L):
        ztile[r, :] = jnp.zeros((L,), jnp.float32)

    def body(j, _):
        pltpu.sync_copy(ztile, acc.at[pl.ds(sid * RPT + j * L, L)])
        return 0

    lax.fori_loop(0, RPT // L, body, 0)


def _copy_out(acc, out_hbm, cid, sid):
    pltpu.sync_copy(acc.at[pl.ds(sid * RPT, RPT)],
                    out_hbm.at[cid, pl.ds(sid * RPT, RPT)])


@functools.partial(
    pl.kernel,
    out_type=jax.ShapeDtypeStruct((NC, NP, L), jnp.float32),
    mesh=_mesh,
    compiler_params=pltpu.CompilerParams(use_tc_tiling_on_sc=False),
    scratch_types=[
        pltpu.VMEM((L, L), jnp.float32),
        pltpu.VMEM((CHUNK,), jnp.int32),
        pltpu.VMEM((CHUNK,), jnp.int32),
        pltpu.VMEM((CHUNK, L), jnp.float32),
        pltpu.VMEM_SHARED((NP, L), jnp.float32),
        pltpu.SemaphoreType.DMA,
    ],
)
def _sc_scatter(y_hbm, row_hbm, col_hbm, out_hbm,
                ztile, ridx, cidx, rows, acc, sem):
    cid = lax.axis_index("c")
    sid = lax.axis_index("s")
    wid = sid * NC + cid
    _zero_fill(ztile, acc, sid)
    plsc.subcore_barrier()
    base = wid * EPW

    def body(i, _):
        off = base + i * CHUNK
        pltpu.sync_copy(row_hbm.at[pl.ds(off, CHUNK)], ridx)
        pltpu.sync_copy(col_hbm.at[pl.ds(off, CHUNK)], cidx)
        pltpu.async_copy(y_hbm.at[ridx], rows, sem).wait()
        pltpu.sync_copy(rows, acc.at[cidx], add=True)
        return 0

    lax.fori_loop(0, NITER, body, 0)
    plsc.subcore_barrier()
    _copy_out(acc, out_hbm, cid, sid)


@functools.partial(
    pl.kernel,
    out_type=jax.ShapeDtypeStruct((NC, NP, L), jnp.float32),
    mesh=_mesh,
    compiler_params=pltpu.CompilerParams(use_tc_tiling_on_sc=False),
    scratch_types=[
        pltpu.VMEM((L, L), jnp.float32),
        pltpu.VMEM((CHUNK,), jnp.int32),
        pltpu.VMEM((CHUNK, L), jnp.float32),
        pltpu.VMEM_SHARED((NP, L), jnp.float32),
    ],
)
def _sc_degree(col_hbm, out_hbm, ztile, cidx, ones, acc):
    cid = lax.axis_index("c")
    sid = lax.axis_index("s")
    wid = sid * NC + cid
    _zero_fill(ztile, acc, sid)
    for r in range(CHUNK):
        ones[r, :] = jnp.ones((L,), jnp.float32)
    plsc.subcore_barrier()
    base = wid * EPW

    def body(i, _):
        off = base + i * CHUNK
        pltpu.sync_copy(col_hbm.at[pl.ds(off, CHUNK)], cidx)
        pltpu.sync_copy(ones, acc.at[cidx], add=True)
        return 0

    lax.fori_loop(0, NITER, body, 0)
    plsc.subcore_barrier()
    _copy_out(acc, out_hbm, cid, sid)


def _tc1_body(x_ref, w0_ref, w1_ref, d0_ref, d1_ref,
              y0_ref, y1s_ref, dis_ref):
    deg = d0_ref[...] + d1_ref[...]
    dis = jnp.where(deg > 0.0, lax.rsqrt(deg), 0.0)
    x = x_ref[...]
    y0_ref[...] = jnp.dot(x, w0_ref[...], preferred_element_type=jnp.float32)
    y1 = jnp.dot(x, w1_ref[...], preferred_element_type=jnp.float32)
    y1s_ref[...] = dis * y1
    dis_ref[...] = dis


def _tc2_body(y0_ref, a0_ref, a1_ref, dis_ref, b1_ref, w0_ref, w1_ref,
              z0_ref, z1s_ref):
    dis = dis_ref[...]
    h = y0_ref[...] + dis * (a0_ref[...] + a1_ref[...]) + b1_ref[...]
    h = jnp.maximum(h, 0.0)
    z0_ref[...] = jnp.dot(h, w0_ref[...], preferred_element_type=jnp.float32)
    z1 = jnp.dot(h, w1_ref[...], preferred_element_type=jnp.float32)
    z1s_ref[...] = dis * z1


def _tc3_body(z0_ref, a0_ref, a1_ref, dis_ref, b2_ref, out_ref):
    o = z0_ref[...] + dis_ref[...] * (a0_ref[...] + a1_ref[...]) + b2_ref[...]
    m = jnp.max(o, axis=1, keepdims=True)
    s = jnp.sum(jnp.exp(o - m), axis=1, keepdims=True)
    out_ref[...] = o - m - jnp.log(s)


def kernel(x, edge_index, W1_0, W1_1, b1, W2_0, W2_1, b2):
    n = x.shape[0]
    row = edge_index[0].astype(jnp.int32)
    col = edge_index[1].astype(jnp.int32)

    degp = _sc_degree(col)
    d0, d1 = degp[0, :n], degp[1, :n]

    y0, y1s, dis = pl.pallas_call(
        _tc1_body,
        out_shape=(
            jax.ShapeDtypeStruct((n, D_HID), jnp.float32),
            jax.ShapeDtypeStruct((n, D_HID), jnp.float32),
            jax.ShapeDtypeStruct((n, D_HID), jnp.float32),
        ),
    )(x, W1_0, W1_1, d0, d1)

    y1s_p = jnp.zeros((NP, D_HID), jnp.float32).at[:n].set(y1s)
    agg1 = _sc_scatter(y1s_p, row, col)

    z0, z1s = pl.pallas_call(
        _tc2_body,
        out_shape=(
            jax.ShapeDtypeStruct((n, D_HID), jnp.float32),
            jax.ShapeDtypeStruct((n, D_HID), jnp.float32),
        ),
    )(y0, agg1[0, :n], agg1[1, :n], dis, b1.reshape(1, D_HID), W2_0, W2_1)

    z1s_p = jnp.zeros((NP, D_HID), jnp.float32).at[:n].set(z1s)
    agg2 = _sc_scatter(z1s_p, row, col)

    out = pl.pallas_call(
        _tc3_body,
        out_shape=jax.ShapeDtypeStruct((n, D_HID), jnp.float32),
    )(z0, agg2[0, :n], agg2[1, :n], dis, b2.reshape(1, D_HID))
    return out


# speedup vs baseline: 24.8558x; 1.8176x over previous
import functools

import jax
import jax.numpy as jnp
from jax import lax
from jax.experimental import pallas as pl
from jax.experimental.pallas import tpu as pltpu
from jax.experimental.pallas import tpu_sc as plsc

N_NODES = 10000
N_EDGES = 320000
D_FEAT = 128
D_HID = 16

NC = 2
NS = 16
NW = NC * NS
L = 16

NP = 10240
RPT = NP // NS
CHUNK = 128
NITER = 80
EPW = NITER * CHUNK
E_PAD = EPW * NW

_mesh = plsc.VectorSubcoreMesh(core_axis_name="c", subcore_axis_name="s")


def _zero_fill(zbuf, acc, sid):
    for r in range(CHUNK):
        zbuf[r, :] = jnp.zeros((L,), jnp.float32)

    def body(j, _):
        pltpu.sync_copy(zbuf, acc.at[pl.ds(sid * RPT + j * CHUNK, CHUNK)])
        return 0

    lax.fori_loop(0, RPT // CHUNK, body, 0)


def _copy_out(acc, out_hbm, cid, sid):
    pltpu.sync_copy(acc.at[pl.ds(sid * RPT, RPT)],
                    out_hbm.at[cid, pl.ds(sid * RPT, RPT)])


@functools.partial(
    pl.kernel,
    out_type=jax.ShapeDtypeStruct((NC, NP, L), jnp.float32),
    mesh=_mesh,
    compiler_params=pltpu.CompilerParams(use_tc_tiling_on_sc=False),
    scratch_types=[
        pltpu.VMEM((NITER, CHUNK), jnp.int32),
        pltpu.VMEM((NITER, CHUNK), jnp.int32),
        pltpu.VMEM((CHUNK, L), jnp.float32),
        pltpu.VMEM((CHUNK, L), jnp.float32),
        pltpu.VMEM_SHARED((NP, L), jnp.float32),
        pltpu.SemaphoreType.DMA,
        pltpu.SemaphoreType.DMA,
    ],
)
def _sc_scatter(y_hbm, row_hbm, col_hbm, out_hbm,
                ridx, cidx, rows0, rows1, acc, sem0, sem1):
    cid = lax.axis_index("c")
    sid = lax.axis_index("s")
    wid = sid * NC + cid
    pltpu.sync_copy(row_hbm.at[wid], ridx)
    pltpu.sync_copy(col_hbm.at[wid], cidx)
    _zero_fill(rows0, acc, sid)
    plsc.subcore_barrier()

    rows = (rows0, rows1)
    sems = (sem0, sem1)
    gather0 = pltpu.async_copy(y_hbm.at[ridx.at[0]], rows0, sem0)

    @pl.loop(0, NITER // 2)
    def _steps(g):
        i0 = g * 2
        for b in range(2):
            i = i0 + b
            nxt = i + 1
            pltpu.make_async_copy(y_hbm.at[ridx.at[i]], rows[b], sems[b]).wait()

            @pl.when(nxt < NITER)
            def _prefetch():
                pltpu.async_copy(y_hbm.at[ridx.at[nxt]], rows[1 - b], sems[1 - b])

            pltpu.sync_copy(rows[b], acc.at[cidx.at[i]], add=True)

    del gather0
    plsc.subcore_barrier()
    _copy_out(acc, out_hbm, cid, sid)


@functools.partial(
    pl.kernel,
    out_type=jax.ShapeDtypeStruct((NC, NP, L), jnp.float32),
    mesh=_mesh,
    compiler_params=pltpu.CompilerParams(use_tc_tiling_on_sc=False),
    scratch_types=[
        pltpu.VMEM((NITER, CHUNK), jnp.int32),
        pltpu.VMEM((CHUNK, L), jnp.float32),
        pltpu.VMEM_SHARED((NP, L), jnp.float32),
    ],
)
def _sc_degree(col_hbm, out_hbm, cidx, ones, acc):
    cid = lax.axis_index("c")
    sid = lax.axis_index("s")
    wid = sid * NC + cid
    pltpu.sync_copy(col_hbm.at[wid], cidx)
    _zero_fill(ones, acc, sid)
    for r in range(CHUNK):
        ones[r, :] = jnp.ones((L,), jnp.float32)
    plsc.subcore_barrier()

    @pl.loop(0, NITER)
    def _steps(i):
        pltpu.sync_copy(ones, acc.at[cidx.at[i]], add=True)

    plsc.subcore_barrier()
    _copy_out(acc, out_hbm, cid, sid)


def _tc_mm1_body(x_ref, w0_ref, w1_ref, y0_ref, y1_ref):
    x = x_ref[...]
    y0_ref[...] = jnp.dot(x, w0_ref[...], preferred_element_type=jnp.float32)
    y1_ref[...] = jnp.dot(x, w1_ref[...], preferred_element_type=jnp.float32)


def _tc_scale_body(y1_ref, d0_ref, d1_ref, y1s_ref, dis_ref):
    deg = d0_ref[...] + d1_ref[...]
    dis = jnp.where(deg > 0.0, lax.rsqrt(deg), 0.0)
    dis_ref[...] = dis
    y1s_ref[pl.ds(0, N_NODES), :] = dis * y1_ref[...]
    y1s_ref[pl.ds(N_NODES, NP - N_NODES), :] = jnp.zeros(
        (NP - N_NODES, D_HID), jnp.float32)


def _tc2_body(y0_ref, a0_ref, a1_ref, dis_ref, b1_ref, w0_ref, w1_ref,
              z0_ref, z1s_ref):
    dis = dis_ref[...]
    h = y0_ref[...] + dis * (a0_ref[...] + a1_ref[...]) + b1_ref[...]
    h = jnp.maximum(h, 0.0)
    z0_ref[...] = jnp.dot(h, w0_ref[...], preferred_element_type=jnp.float32)
    z1 = jnp.dot(h, w1_ref[...], preferred_element_type=jnp.float32)
    z1s_ref[pl.ds(0, N_NODES), :] = dis * z1
    z1s_ref[pl.ds(N_NODES, NP - N_NODES), :] = jnp.zeros(
        (NP - N_NODES, D_HID), jnp.float32)


def _tc3_body(z0_ref, a0_ref, a1_ref, dis_ref, b2_ref, out_ref):
    o = z0_ref[...] + dis_ref[...] * (a0_ref[...] + a1_ref[...]) + b2_ref[...]
    m = jnp.max(o, axis=1, keepdims=True)
    s = jnp.sum(jnp.exp(o - m), axis=1, keepdims=True)
    out_ref[...] = o - m - jnp.log(s)


def kernel(x, edge_index, W1_0, W1_1, b1, W2_0, W2_1, b2):
    n = x.shape[0]
    row = edge_index[0].astype(jnp.int32)
    col = edge_index[1].astype(jnp.int32)
    npad = E_PAD - N_EDGES
    row3 = jnp.concatenate(
        [row, jnp.zeros((npad,), jnp.int32)]).reshape(NW, NITER, CHUNK)
    col3 = jnp.concatenate(
        [col, jnp.full((npad,), NP - 1, jnp.int32)]).reshape(NW, NITER, CHUNK)

    degp = _sc_degree(col3)
    y0, y1 = pl.pallas_call(
        _tc_mm1_body,
        out_shape=(
            jax.ShapeDtypeStruct((n, D_HID), jnp.float32),
            jax.ShapeDtypeStruct((n, D_HID), jnp.float32),
        ),
    )(x, W1_0, W1_1)

    y1s_p, dis = pl.pallas_call(
        _tc_scale_body,
        out_shape=(
            jax.ShapeDtypeStruct((NP, D_HID), jnp.float32),
            jax.ShapeDtypeStruct((n, D_HID), jnp.float32),
        ),
    )(y1, degp[0, :n], degp[1, :n])

    agg1 = _sc_scatter(y1s_p, row3, col3)

    z0, z1s_p = pl.pallas_call(
        _tc2_body,
        out_shape=(
            jax.ShapeDtypeStruct((n, D_HID), jnp.float32),
            jax.ShapeDtypeStruct((NP, D_HID), jnp.float32),
        ),
    )(y0, agg1[0, :n], agg1[1, :n], dis, b1.reshape(1, D_HID), W2_0, W2_1)

    agg2 = _sc_scatter(z1s_p, row3, col3)

    out = pl.pallas_call(
        _tc3_body,
        out_shape=jax.ShapeDtypeStruct((n, D_HID), jnp.float32),
    )(z0, agg2[0, :n], agg2[1, :n], dis, b2.reshape(1, D_HID))
    return out
